# Initial kernel scaffold; baseline (speedup 1.0000x reference)
import jax
import jax.numpy as jnp
from jax.experimental import pallas as pl


def kernel(X_w_indices, X_d, y, z, n, W, b):
    raise NotImplementedError("write your pallas kernel here")


# speedup vs baseline: 1.6522x; 1.6522x over previous
import functools

import jax
import jax.numpy as jnp
from jax import lax
from jax.experimental import pallas as pl
from jax.experimental.pallas import tpu as pltpu
from jax.experimental.pallas import tpu_sc as plsc

_ALPHA = 0.1
_BETA = 1.0
_L1 = 1.0
_L2 = 1.0
_D = 1000000
_B = 16384
_F_W = 100
_DP = 1 << 20

_NC, _NS = 2, 16
_NW = _NC * _NS
_CHUNK = (_B * _F_W) // _NW


def _wtab_body(z_ref, n_ref, o_ref):
    zv = z_ref[...]
    nv = n_ref[...]
    sign = jnp.where(zv < 0, -1.0, 1.0)
    denom = (_BETA + jnp.sqrt(nv)) / _ALPHA + _L2
    o_ref[...] = jnp.where(sign * zv <= _L1, 0.0, (sign * _L1 - zv) / denom)


_wtab = pl.pallas_call(
    _wtab_body,
    grid=(8,),
    in_specs=[
        pl.BlockSpec((128, 1024), lambda i: (i, 0)),
        pl.BlockSpec((128, 1024), lambda i: (i, 0)),
    ],
    out_specs=pl.BlockSpec((128, 1024), lambda i: (i, 0)),
    out_shape=jax.ShapeDtypeStruct((1024, 1024), jnp.float32),
)


@functools.cache
def _build_gather_sc():
    mesh = plsc.VectorSubcoreMesh(core_axis_name="c", subcore_axis_name="s")

    @functools.partial(
        pl.kernel,
        out_type=jax.ShapeDtypeStruct((_B * _F_W,), jnp.float32),
        mesh=mesh,
        scratch_types=[
            pltpu.VMEM((_CHUNK,), jnp.int32),
            pltpu.VMEM((_CHUNK,), jnp.float32),
            pltpu.SemaphoreType.DMA,
        ],
    )
    def _gather_sc(idx_hbm, tab_hbm, out_hbm, idx_v, val_v, sem):
        wid = lax.axis_index("s") * _NC + lax.axis_index("c")
        base = wid * _CHUNK
        pltpu.sync_copy(idx_hbm.at[pl.ds(base, _CHUNK)], idx_v)
        pltpu.async_copy(tab_hbm.at[idx_v], val_v, sem).wait()
        pltpu.sync_copy(val_v, out_hbm.at[pl.ds(base, _CHUNK)])

    return _gather_sc


def _finish_body(b_ref, v_ref, o_ref):
    s = jnp.sum(v_ref[...], axis=1)
    wide_z = jnp.clip(s, -35.0, 35.0) + b_ref[0]
    o_ref[...] = 1.0 / (1.0 + jnp.exp(-wide_z))


_finish = pl.pallas_call(
    _finish_body,
    grid=(8,),
    in_specs=[
        pl.BlockSpec(memory_space=pltpu.SMEM),
        pl.BlockSpec((_B // 8, _F_W), lambda i: (i, 0)),
    ],
    out_specs=pl.BlockSpec((_B // 8,), lambda i: (i,)),
    out_shape=jax.ShapeDtypeStruct((_B,), jnp.float32),
)


def kernel(X_w_indices, X_d, y, z, n, W, b):
    zp = jnp.concatenate([z, jnp.zeros((_DP - _D,), jnp.float32)])
    npad = jnp.concatenate([n, jnp.ones((_DP - _D,), jnp.float32)])
    wtab = _wtab(zp.reshape(1024, 1024), npad.reshape(1024, 1024)).reshape(_DP)
    idx_flat = X_w_indices.reshape(-1)
    vals = _build_gather_sc()(idx_flat, wtab)
    y_pred = _finish(b, vals.reshape(_B, _F_W))
    return y_pred.reshape(_B, 1)


# speedup vs baseline: 1.9388x; 1.1734x over previous
import functools

import jax
import jax.numpy as jnp
from jax import lax
from jax.experimental import pallas as pl
from jax.experimental.pallas import tpu as pltpu
from jax.experimental.pallas import tpu_sc as plsc

_ALPHA = 0.1
_BETA = 1.0
_L1 = 1.0
_L2 = 1.0
_D = 1000000
_B = 16384
_F_W = 100

_NC, _NS = 2, 16
_NW = _NC * _NS
_ROWS = _B // _NW
_CHUNK = _ROWS * _F_W


def _wtab_body(z_ref, n_ref, o_ref):
    zv = z_ref[...]
    nv = n_ref[...]
    sign = jnp.where(zv < 0, -1.0, 1.0)
    denom = (_BETA + jnp.sqrt(nv)) / _ALPHA + _L2
    o_ref[...] = jnp.where(sign * zv <= _L1, 0.0, (sign * _L1 - zv) / denom)


_wtab = pl.pallas_call(
    _wtab_body,
    out_shape=jax.ShapeDtypeStruct((_D,), jnp.float32),
)


@functools.cache
def _build_gather_sc():
    mesh = plsc.VectorSubcoreMesh(core_axis_name="c", subcore_axis_name="s")

    @functools.partial(
        pl.kernel,
        out_type=jax.ShapeDtypeStruct((_B,), jnp.float32),
        mesh=mesh,
        scratch_types=[
            pltpu.VMEM((_CHUNK,), jnp.int32),
            pltpu.VMEM((_CHUNK,), jnp.float32),
            pltpu.VMEM((_ROWS,), jnp.float32),
            pltpu.VMEM((16,), jnp.float32),
            pltpu.SemaphoreType.DMA,
        ],
    )
    def _gather_sc(idx_hbm, tab_hbm, b_hbm, out_hbm, idx_v, val_v,
                   out_v, b_v, sem):
        wid = lax.axis_index("s") * _NC + lax.axis_index("c")
        base = wid * _CHUNK
        pltpu.sync_copy(b_hbm, b_v)
        pltpu.sync_copy(idx_hbm.at[pl.ds(base, _CHUNK)], idx_v)
        pltpu.async_copy(tab_hbm.at[idx_v], val_v, sem).wait()

        bvec = b_v[...]
        zero16 = jnp.zeros((16,), jnp.float32)
        for g in range(_ROWS // 16):
            off = g * 16

            def _col(j, a):
                return a + val_v[pl.ds(j * _ROWS + off, 16)]

            tot = lax.fori_loop(0, _F_W, _col, zero16)
            x = jnp.maximum(jnp.minimum(tot, 35.0), -35.0) + bvec
            out_v[pl.ds(g * 16, 16)] = 1.0 / (1.0 + jnp.exp(-x))

        pltpu.sync_copy(out_v, out_hbm.at[pl.ds(wid * _ROWS, _ROWS)])

    return _gather_sc


def kernel(X_w_indices, X_d, y, z, n, W, b):
    wtab = _wtab(z, n)
    idx_t = X_w_indices.reshape(_NW, _ROWS, _F_W).transpose(0, 2, 1)
    bb = jnp.broadcast_to(b, (16,))
    y_pred = _build_gather_sc()(idx_t.reshape(-1), wtab, bb)
    return y_pred.reshape(_B, 1)


# speedup vs baseline: 2.0070x; 1.0352x over previous
import functools

import jax
import jax.numpy as jnp
from jax import lax
from jax.experimental import pallas as pl
from jax.experimental.pallas import tpu as pltpu
from jax.experimental.pallas import tpu_sc as plsc

_ALPHA = 0.1
_BETA = 1.0
_L1 = 1.0
_L2 = 1.0
_D = 1000000
_B = 16384
_F_W = 100

_NC, _NS = 2, 16
_NW = _NC * _NS
_ROWS = _B // _NW
_FCH = 25
_NCHUNK = _F_W // _FCH
_CELEMS = _FCH * _ROWS


def _wtab_body(z_ref, n_ref, o_ref):
    zv = z_ref[...]
    nv = n_ref[...]
    sign = jnp.where(zv < 0, -1.0, 1.0)
    denom = (_BETA + jnp.sqrt(nv)) / _ALPHA + _L2
    o_ref[...] = jnp.where(sign * zv <= _L1, 0.0, (sign * _L1 - zv) / denom)


_wtab = pl.pallas_call(
    _wtab_body,
    out_shape=jax.ShapeDtypeStruct((_D,), jnp.float32),
)


@functools.cache
def _build_gather_sc():
    mesh = plsc.VectorSubcoreMesh(core_axis_name="c", subcore_axis_name="s")

    @functools.partial(
        pl.kernel,
        out_type=jax.ShapeDtypeStruct((_B,), jnp.float32),
        mesh=mesh,
        scratch_types=[
            pltpu.VMEM((_F_W * _ROWS,), jnp.int32),
            pltpu.VMEM((_CELEMS,), jnp.float32),
            pltpu.VMEM((_CELEMS,), jnp.float32),
            pltpu.VMEM((_ROWS,), jnp.float32),
            pltpu.VMEM((16,), jnp.float32),
            pltpu.SemaphoreType.DMA,
            pltpu.SemaphoreType.DMA,
            pltpu.SemaphoreType.DMA,
        ],
    )
    def _gather_sc(idx_hbm, tab_hbm, b_hbm, out_hbm, idx_v, val_a, val_b,
                   acc_v, b_v, sem_i, sem_a, sem_b):
        wid = lax.axis_index("s") * _NC + lax.axis_index("c")
        pltpu.sync_copy(b_hbm, b_v)
        idx_dmas = [
            pltpu.async_copy(idx_hbm.at[wid, j],
                             idx_v.at[pl.ds(j * _ROWS, _ROWS)], sem_i)
            for j in range(_F_W)
        ]
        for h in idx_dmas:
            h.wait()

        bufs = (val_a, val_b)
        sems = (sem_a, sem_b)

        def _fire(c):
            buf, sem = bufs[c % 2], sems[c % 2]
            return [
                pltpu.async_copy(
                    tab_hbm.at[idx_v.at[pl.ds((c * _FCH + j) * _ROWS, _ROWS)]],
                    buf.at[pl.ds(j * _ROWS, _ROWS)], sem)
                for j in range(_FCH)
            ]

        def _reduce(c):
            buf = bufs[c % 2]
            for g in range(_ROWS // 16):
                off = g * 16
                a0 = (jnp.zeros((16,), jnp.float32) if c == 0
                      else acc_v[pl.ds(off, 16)])

                def _col(j, a):
                    return a + buf[pl.ds(j * _ROWS + off, 16)]

                acc_v[pl.ds(off, 16)] = lax.fori_loop(0, _FCH, _col, a0)

        pending = _fire(0)
        for c in range(_NCHUNK):
            nxt = _fire(c + 1) if c + 1 < _NCHUNK else []
            for h in pending:
                h.wait()
            pending = nxt
            _reduce(c)

        bvec = b_v[...]
        for g in range(_ROWS // 16):
            tot = acc_v[pl.ds(g * 16, 16)]
            x = jnp.maximum(jnp.minimum(tot, 35.0), -35.0) + bvec
            acc_v[pl.ds(g * 16, 16)] = 1.0 / (1.0 + jnp.exp(-x))

        pltpu.sync_copy(acc_v, out_hbm.at[pl.ds(wid * _ROWS, _ROWS)])

    return _gather_sc


def kernel(X_w_indices, X_d, y, z, n, W, b):
    wtab = _wtab(z, n)
    idx_t = X_w_indices.reshape(_NW, _ROWS, _F_W).transpose(0, 2, 1)
    bb = jnp.broadcast_to(b, (16,))
    y_pred = _build_gather_sc()(idx_t, wtab, bb)
    return y_pred.reshape(_B, 1)


# speedup vs baseline: 2.5263x; 1.2588x over previous
import functools

import jax
import jax.numpy as jnp
from jax import lax
from jax.experimental import pallas as pl
from jax.experimental.pallas import tpu as pltpu
from jax.experimental.pallas import tpu_sc as plsc

_ALPHA = 0.1
_BETA = 1.0
_L1 = 1.0
_L2 = 1.0
_D = 1000000
_B = 16384
_F_W = 100

_NC, _NS = 2, 16
_NW = _NC * _NS
_ROWS = _B // _NW
_FCH = 25
_NCHUNK = _F_W // _FCH
_CELEMS = _FCH * _ROWS


def _wtab_body(z_ref, n_ref, o_ref):
    zv = z_ref[...]
    nv = n_ref[...]
    sign = jnp.where(zv < 0, -1.0, 1.0)
    denom = (_BETA + jnp.sqrt(nv)) / _ALPHA + _L2
    o_ref[...] = jnp.where(sign * zv <= _L1, 0.0, (sign * _L1 - zv) / denom)


_wtab = pl.pallas_call(
    _wtab_body,
    out_shape=jax.ShapeDtypeStruct((_D,), jnp.float32),
)


@functools.cache
def _build_gather_sc():
    mesh = plsc.VectorSubcoreMesh(core_axis_name="c", subcore_axis_name="s")

    @functools.partial(
        pl.kernel,
        out_type=jax.ShapeDtypeStruct((_B,), jnp.float32),
        mesh=mesh,
        scratch_types=[
            pltpu.VMEM((_F_W * _ROWS,), jnp.int32),
            pltpu.VMEM((_CELEMS,), jnp.float32),
            pltpu.VMEM((_CELEMS,), jnp.float32),
            pltpu.VMEM((_ROWS,), jnp.float32),
            pltpu.VMEM((16,), jnp.float32),
            pltpu.SemaphoreType.DMA,
            pltpu.SemaphoreType.DMA,
            pltpu.SemaphoreType.DMA,
        ],
    )
    def _gather_sc(idx_hbm, tab_hbm, b_hbm, out_hbm, idx_v, val_a, val_b,
                   acc_v, b_v, sem_i, sem_a, sem_b):
        wid = lax.axis_index("s") * _NC + lax.axis_index("c")
        pltpu.sync_copy(b_hbm, b_v)
        idx_dmas = [
            pltpu.async_copy(idx_hbm.at[j, pl.ds(wid * _ROWS, _ROWS)],
                             idx_v.at[pl.ds(j * _ROWS, _ROWS)], sem_i)
            for j in range(_F_W)
        ]
        for h in idx_dmas:
            h.wait()

        bufs = (val_a, val_b)
        sems = (sem_a, sem_b)

        def _fire(c):
            buf, sem = bufs[c % 2], sems[c % 2]
            return [
                pltpu.async_copy(
                    tab_hbm.at[idx_v.at[pl.ds((c * _FCH + j) * _ROWS, _ROWS)]],
                    buf.at[pl.ds(j * _ROWS, _ROWS)], sem)
                for j in range(_FCH)
            ]

        def _reduce(c):
            buf = bufs[c % 2]
            for g in range(_ROWS // 16):
                off = g * 16
                a0 = (jnp.zeros((16,), jnp.float32) if c == 0
                      else acc_v[pl.ds(off, 16)])

                def _col(j, a):
                    return a + buf[pl.ds(j * _ROWS + off, 16)]

                acc_v[pl.ds(off, 16)] = lax.fori_loop(0, _FCH, _col, a0)

        pending = _fire(0)
        for c in range(_NCHUNK):
            nxt = _fire(c + 1) if c + 1 < _NCHUNK else []
            for h in pending:
                h.wait()
            pending = nxt
            _reduce(c)

        bvec = b_v[...]
        for g in range(_ROWS // 16):
            tot = acc_v[pl.ds(g * 16, 16)]
            x = jnp.maximum(jnp.minimum(tot, 35.0), -35.0) + bvec
            acc_v[pl.ds(g * 16, 16)] = 1.0 / (1.0 + jnp.exp(-x))

        pltpu.sync_copy(acc_v, out_hbm.at[pl.ds(wid * _ROWS, _ROWS)])

    return _gather_sc


def kernel(X_w_indices, X_d, y, z, n, W, b):
    wtab = _wtab(z, n)
    idx_t = X_w_indices.T
    bb = jnp.broadcast_to(b, (16,))
    y_pred = _build_gather_sc()(idx_t, wtab, bb)
    return y_pred.reshape(_B, 1)


# speedup vs baseline: 3.3060x; 1.3086x over previous
import functools

import jax
import jax.numpy as jnp
from jax import lax
from jax.experimental import pallas as pl
from jax.experimental.pallas import tpu as pltpu
from jax.experimental.pallas import tpu_sc as plsc

_ALPHA = 0.1
_BETA = 1.0
_L1 = 1.0
_L2 = 1.0
_D = 1000000
_B = 16384
_F_W = 100

_NC, _NS = 2, 16
_NW = _NC * _NS
_ROWS = _B // _NW
_FCH = 20
_NCHUNK = _F_W // _FCH
_CELEMS = _FCH * _ROWS
_FROM_SPM = (False, True, False, True, True)
_SH = 62504
_BNC = 10240


def _wtab_body(z_ref, n_ref, o_ref):
    zv = z_ref[...]
    nv = n_ref[...]
    sign = jnp.where(zv < 0, -1.0, 1.0)
    denom = (_BETA + jnp.sqrt(nv)) / _ALPHA + _L2
    o_ref[...] = jnp.where(sign * zv <= _L1, 0.0, (sign * _L1 - zv) / denom)


_wtab = pl.pallas_call(
    _wtab_body,
    out_shape=jax.ShapeDtypeStruct((_D,), jnp.float32),
)


@functools.cache
def _build_gather_sc():
    mesh = plsc.VectorSubcoreMesh(core_axis_name="c", subcore_axis_name="s")

    @functools.partial(
        pl.kernel,
        out_type=jax.ShapeDtypeStruct((_B,), jnp.float32),
        mesh=mesh,
        scratch_types=[
            pltpu.VMEM((_CELEMS,), jnp.int32),
            pltpu.VMEM((_CELEMS,), jnp.int32),
            pltpu.VMEM((_CELEMS,), jnp.float32),
            pltpu.VMEM((_CELEMS,), jnp.float32),
            pltpu.VMEM((_ROWS,), jnp.float32),
            pltpu.VMEM((16,), jnp.float32),
            pltpu.VMEM_SHARED((_D,), jnp.float32),
            pltpu.SemaphoreType.DMA,
            pltpu.SemaphoreType.DMA,
            pltpu.SemaphoreType.DMA,
            pltpu.SemaphoreType.DMA,
        ],
    )
    def _gather_sc(idx_hbm, tab_hbm, b_hbm, out_hbm, idx_a, idx_b,
                   val_a, val_b, acc_v, b_v, spm,
                   sem_ia, sem_ib, sem_a, sem_b):
        wid = lax.axis_index("s") * _NC + lax.axis_index("c")
        sid = lax.axis_index("s")
        pltpu.sync_copy(b_hbm, b_v)

        ibufs = (idx_a, idx_b)
        isems = (sem_ia, sem_ib)
        vbufs = (val_a, val_b)
        vsems = (sem_a, sem_b)

        def _fire_idx(c):
            buf, sem = ibufs[c % 2], isems[c % 2]
            return [
                pltpu.async_copy(
                    idx_hbm.at[c * _FCH + j, pl.ds(wid * _ROWS, _ROWS)],
                    buf.at[pl.ds(j * _ROWS, _ROWS)], sem)
                for j in range(_FCH)
            ]

        soff = sid * _SH

        def _stage(shard):
            for k in range((shard + _BNC - 1) // _BNC):
                csz = min(_BNC, shard - k * _BNC)
                pltpu.sync_copy(tab_hbm.at[pl.ds(soff + k * _BNC, csz)],
                                val_a.at[pl.ds(0, csz)])
                pltpu.sync_copy(val_a.at[pl.ds(0, csz)],
                                spm.at[pl.ds(soff + k * _BNC, csz)])

        idx_pending = [_fire_idx(0), _fire_idx(1)]

        @pl.when(sid < _NS - 1)
        def _stage_full():
            _stage(_SH)

        @pl.when(sid == _NS - 1)
        def _stage_last():
            _stage(_D - (_NS - 1) * _SH)

        plsc.subcore_barrier()

        def _fire_g(c):
            buf, sem = vbufs[c % 2], vsems[c % 2]
            ibuf = ibufs[c % 2]
            src = spm if _FROM_SPM[c] else tab_hbm
            return [
                pltpu.async_copy(
                    src.at[ibuf.at[pl.ds(j * _ROWS, _ROWS)]],
                    buf.at[pl.ds(j * _ROWS, _ROWS)], sem)
                for j in range(_FCH)
            ]

        def _reduce(c):
            buf = vbufs[c % 2]
            for g in range(_ROWS // 16):
                off = g * 16
                a0 = (jnp.zeros((16,), jnp.float32) if c == 0
                      else acc_v[pl.ds(off, 16)])

                def _col(j, a):
                    return a + buf[pl.ds(j * _ROWS + off, 16)]

                acc_v[pl.ds(off, 16)] = lax.fori_loop(0, _FCH, _col, a0)

        g_pending = [None] * _NCHUNK
        for c in range(_NCHUNK):
            for h in idx_pending[c]:
                h.wait()
            g_pending[c] = _fire_g(c)
            if c >= 1:
                for h in g_pending[c - 1]:
                    h.wait()
                if c + 1 < _NCHUNK:
                    idx_pending.append(_fire_idx(c + 1))
                _reduce(c - 1)
        for h in g_pending[_NCHUNK - 1]:
            h.wait()
        _reduce(_NCHUNK - 1)

        bvec = b_v[...]
        for g in range(_ROWS // 16):
            tot = acc_v[pl.ds(g * 16, 16)]
            x = jnp.maximum(jnp.minimum(tot, 35.0), -35.0) + bvec
            acc_v[pl.ds(g * 16, 16)] = 1.0 / (1.0 + jnp.exp(-x))

        pltpu.sync_copy(acc_v, out_hbm.at[pl.ds(wid * _ROWS, _ROWS)])

    return _gather_sc


def kernel(X_w_indices, X_d, y, z, n, W, b):
    wtab = _wtab(z, n)
    idx_t = X_w_indices.T
    bb = jnp.broadcast_to(b, (16,))
    y_pred = _build_gather_sc()(idx_t, wtab, bb)
    return y_pred.reshape(_B, 1)


# speedup vs baseline: 4.1383x; 1.2518x over previous
import functools

import jax
import jax.numpy as jnp
from jax import lax
from jax.experimental import pallas as pl
from jax.experimental.pallas import tpu as pltpu
from jax.experimental.pallas import tpu_sc as plsc

_ALPHA = 0.1
_BETA = 1.0
_L1 = 1.0
_L2 = 1.0
_D = 1000000
_B = 16384
_F_W = 100

_NC, _NS = 2, 16
_NW = _NC * _NS
_ROWS = _B // _NW
_FCH = 20
_NCHUNK = _F_W // _FCH
_CELEMS = _FCH * _ROWS
_FROM_SPM = (True, True, True, True, True)
_SH = 62504
_BNC = 10240


def _wtab_body(z_ref, n_ref, o_ref):
    zv = z_ref[...]
    nv = n_ref[...]
    sign = jnp.where(zv < 0, -1.0, 1.0)
    denom = (_BETA + jnp.sqrt(nv)) / _ALPHA + _L2
    o_ref[...] = jnp.where(sign * zv <= _L1, 0.0, (sign * _L1 - zv) / denom)


_wtab = pl.pallas_call(
    _wtab_body,
    out_shape=jax.ShapeDtypeStruct((_D,), jnp.float32),
)


@functools.cache
def _build_gather_sc():
    mesh = plsc.VectorSubcoreMesh(core_axis_name="c", subcore_axis_name="s")

    @functools.partial(
        pl.kernel,
        out_type=jax.ShapeDtypeStruct((_B,), jnp.float32),
        mesh=mesh,
        scratch_types=[
            pltpu.VMEM((_CELEMS,), jnp.int32),
            pltpu.VMEM((_CELEMS,), jnp.int32),
            pltpu.VMEM((_CELEMS,), jnp.float32),
            pltpu.VMEM((_CELEMS,), jnp.float32),
            pltpu.VMEM((_ROWS,), jnp.float32),
            pltpu.VMEM((16,), jnp.float32),
            pltpu.VMEM_SHARED((_D,), jnp.float32),
            pltpu.SemaphoreType.DMA,
            pltpu.SemaphoreType.DMA,
            pltpu.SemaphoreType.DMA,
            pltpu.SemaphoreType.DMA,
        ],
    )
    def _gather_sc(idx_hbm, tab_hbm, b_hbm, out_hbm, idx_a, idx_b,
                   val_a, val_b, acc_v, b_v, spm,
                   sem_ia, sem_ib, sem_a, sem_b):
        wid = lax.axis_index("s") * _NC + lax.axis_index("c")
        sid = lax.axis_index("s")
        pltpu.sync_copy(b_hbm, b_v)

        ibufs = (idx_a, idx_b)
        isems = (sem_ia, sem_ib)
        vbufs = (val_a, val_b)
        vsems = (sem_a, sem_b)

        def _fire_idx(c):
            buf, sem = ibufs[c % 2], isems[c % 2]
            return [
                pltpu.async_copy(
                    idx_hbm.at[c * _FCH + j, pl.ds(wid * _ROWS, _ROWS)],
                    buf.at[pl.ds(j * _ROWS, _ROWS)], sem)
                for j in range(_FCH)
            ]

        soff = sid * _SH

        def _stage(shard):
            for k in range((shard + _BNC - 1) // _BNC):
                csz = min(_BNC, shard - k * _BNC)
                pltpu.sync_copy(tab_hbm.at[pl.ds(soff + k * _BNC, csz)],
                                val_a.at[pl.ds(0, csz)])
                pltpu.sync_copy(val_a.at[pl.ds(0, csz)],
                                spm.at[pl.ds(soff + k * _BNC, csz)])

        idx_pending = [_fire_idx(0), _fire_idx(1)]

        @pl.when(sid < _NS - 1)
        def _stage_full():
            _stage(_SH)

        @pl.when(sid == _NS - 1)
        def _stage_last():
            _stage(_D - (_NS - 1) * _SH)

        plsc.subcore_barrier()

        def _fire_g(c):
            buf, sem = vbufs[c % 2], vsems[c % 2]
            ibuf = ibufs[c % 2]
            src = spm if _FROM_SPM[c] else tab_hbm
            return [
                pltpu.async_copy(
                    src.at[ibuf.at[pl.ds(j * _ROWS, _ROWS)]],
                    buf.at[pl.ds(j * _ROWS, _ROWS)], sem)
                for j in range(_FCH)
            ]

        def _reduce(c):
            buf = vbufs[c % 2]
            for g in range(_ROWS // 16):
                off = g * 16
                a0 = (jnp.zeros((16,), jnp.float32) if c == 0
                      else acc_v[pl.ds(off, 16)])

                def _col(j, a):
                    return a + buf[pl.ds(j * _ROWS + off, 16)]

                acc_v[pl.ds(off, 16)] = lax.fori_loop(0, _FCH, _col, a0)

        g_pending = [None] * _NCHUNK
        for c in range(_NCHUNK):
            for h in idx_pending[c]:
                h.wait()
            g_pending[c] = _fire_g(c)
            if c >= 1:
                for h in g_pending[c - 1]:
                    h.wait()
                if c + 1 < _NCHUNK:
                    idx_pending.append(_fire_idx(c + 1))
                _reduce(c - 1)
        for h in g_pending[_NCHUNK - 1]:
            h.wait()
        _reduce(_NCHUNK - 1)

        bvec = b_v[...]
        for g in range(_ROWS // 16):
            tot = acc_v[pl.ds(g * 16, 16)]
            x = jnp.maximum(jnp.minimum(tot, 35.0), -35.0) + bvec
            acc_v[pl.ds(g * 16, 16)] = 1.0 / (1.0 + jnp.exp(-x))

        pltpu.sync_copy(acc_v, out_hbm.at[pl.ds(wid * _ROWS, _ROWS)])

    return _gather_sc


def kernel(X_w_indices, X_d, y, z, n, W, b):
    wtab = _wtab(z, n)
    idx_t = X_w_indices.T
    bb = jnp.broadcast_to(b, (16,))
    y_pred = _build_gather_sc()(idx_t, wtab, bb)
    return y_pred.reshape(_B, 1)


# speedup vs baseline: 4.7720x; 1.1531x over previous
import functools

import jax
import jax.numpy as jnp
from jax import lax
from jax.experimental import pallas as pl
from jax.experimental.pallas import tpu as pltpu
from jax.experimental.pallas import tpu_sc as plsc

_ALPHA = 0.1
_BETA = 1.0
_L1 = 1.0
_L2 = 1.0
_D = 1000000
_B = 16384
_F_W = 100

_NC, _NS = 2, 16
_NW = _NC * _NS
_ROWS = _B // _NW
_FCH = 20
_NCHUNK = _F_W // _FCH
_CELEMS = _FCH * _ROWS
_FROM_SPM = (True, True, True, True, True)
_SH = 62504
_BNC = 10240


def _wtab_body(z_ref, n_ref, o_ref):
    zv = z_ref[...]
    nv = n_ref[...]
    sign = jnp.where(zv < 0, -1.0, 1.0)
    denom = (_BETA + jnp.sqrt(nv)) / _ALPHA + _L2
    o_ref[...] = jnp.where(sign * zv <= _L1, 0.0, (sign * _L1 - zv) / denom)


_wtab = pl.pallas_call(
    _wtab_body,
    out_shape=jax.ShapeDtypeStruct((_D,), jnp.float32),
)


@functools.cache
def _build_gather_sc():
    mesh = plsc.VectorSubcoreMesh(core_axis_name="c", subcore_axis_name="s")

    @functools.partial(
        pl.kernel,
        out_type=jax.ShapeDtypeStruct((_B,), jnp.float32),
        mesh=mesh,
        scratch_types=[
            pltpu.VMEM((_CELEMS,), jnp.int32),
            pltpu.VMEM((_CELEMS,), jnp.int32),
            pltpu.VMEM((_CELEMS,), jnp.float32),
            pltpu.VMEM((_CELEMS,), jnp.float32),
            pltpu.VMEM((_ROWS,), jnp.float32),
            pltpu.VMEM((16,), jnp.float32),
            pltpu.VMEM_SHARED((_D,), jnp.float32),
            pltpu.SemaphoreType.DMA,
            pltpu.SemaphoreType.DMA,
            pltpu.SemaphoreType.DMA,
            pltpu.SemaphoreType.DMA,
        ],
    )
    def _gather_sc(idx_hbm, tab_hbm, b_hbm, out_hbm, idx_a, idx_b,
                   val_a, val_b, acc_v, b_v, spm,
                   sem_ia, sem_ib, sem_a, sem_b):
        wid = lax.axis_index("s") * _NC + lax.axis_index("c")
        sid = lax.axis_index("s")
        pltpu.sync_copy(b_hbm, b_v)

        ibufs = (idx_a, idx_b)
        isems = (sem_ia, sem_ib)
        vbufs = (val_a, val_b)
        vsems = (sem_a, sem_b)

        def _fire_idx(c):
            buf, sem = ibufs[c % 2], isems[c % 2]
            return [
                pltpu.async_copy(
                    idx_hbm.at[c * _FCH + j, pl.ds(wid * _ROWS, _ROWS)],
                    buf.at[pl.ds(j * _ROWS, _ROWS)], sem)
                for j in range(_FCH)
            ]

        soff = sid * _SH

        def _stage(shard):
            nchunks = (shard + _BNC - 1) // _BNC
            szs = [min(_BNC, shard - k * _BNC) for k in range(nchunks)]
            hs = [None] * nchunks
            hs[0] = pltpu.async_copy(tab_hbm.at[pl.ds(soff, szs[0])],
                                     val_a.at[pl.ds(0, szs[0])], sem_a)
            for k in range(nchunks):
                buf = (val_a, val_b)[k % 2]
                if k + 1 < nchunks:
                    nbuf = (val_a, val_b)[(k + 1) % 2]
                    hs[k + 1] = pltpu.async_copy(
                        tab_hbm.at[pl.ds(soff + (k + 1) * _BNC, szs[k + 1])],
                        nbuf.at[pl.ds(0, szs[k + 1])],
                        (sem_a, sem_b)[(k + 1) % 2])
                hs[k].wait()
                pltpu.sync_copy(buf.at[pl.ds(0, szs[k])],
                                spm.at[pl.ds(soff + k * _BNC, szs[k])])

        idx_pending = [_fire_idx(0), _fire_idx(1)]

        @pl.when(sid < _NS - 1)
        def _stage_full():
            _stage(_SH)

        @pl.when(sid == _NS - 1)
        def _stage_last():
            _stage(_D - (_NS - 1) * _SH)

        plsc.subcore_barrier()

        def _fire_g(c):
            buf, sem = vbufs[c % 2], vsems[c % 2]
            ibuf = ibufs[c % 2]
            src = spm if _FROM_SPM[c] else tab_hbm
            return [
                pltpu.async_copy(
                    src.at[ibuf.at[pl.ds(j * _ROWS, _ROWS)]],
                    buf.at[pl.ds(j * _ROWS, _ROWS)], sem)
                for j in range(_FCH)
            ]

        def _reduce(c):
            buf = vbufs[c % 2]

            def _group(g, _):
                off = g * 16
                a = (jnp.zeros((16,), jnp.float32) if c == 0
                     else acc_v[pl.ds(off, 16)])
                for j in range(_FCH):
                    a = a + buf[pl.ds(j * _ROWS + off, 16)]
                acc_v[pl.ds(off, 16)] = a
                return 0

            lax.fori_loop(0, _ROWS // 16, _group, 0)

        g_pending = [None] * _NCHUNK
        for c in range(_NCHUNK):
            for h in idx_pending[c]:
                h.wait()
            g_pending[c] = _fire_g(c)
            if c >= 1:
                for h in g_pending[c - 1]:
                    h.wait()
                if c + 1 < _NCHUNK:
                    idx_pending.append(_fire_idx(c + 1))
                _reduce(c - 1)
        for h in g_pending[_NCHUNK - 1]:
            h.wait()
        _reduce(_NCHUNK - 1)

        bvec = b_v[...]
        for g in range(_ROWS // 16):
            tot = acc_v[pl.ds(g * 16, 16)]
            x = jnp.maximum(jnp.minimum(tot, 35.0), -35.0) + bvec
            acc_v[pl.ds(g * 16, 16)] = 1.0 / (1.0 + jnp.exp(-x))

        pltpu.sync_copy(acc_v, out_hbm.at[pl.ds(wid * _ROWS, _ROWS)])

    return _gather_sc


def kernel(X_w_indices, X_d, y, z, n, W, b):
    wtab = _wtab(z, n)
    idx_t = X_w_indices.T
    bb = jnp.broadcast_to(b, (16,))
    y_pred = _build_gather_sc()(idx_t, wtab, bb)
    return y_pred.reshape(_B, 1)
